# baseline (speedup 1.0000x reference)
import functools
import math

import jax
import jax.numpy as jnp
from jax import lax
from jax.experimental import pallas as pl
from jax.experimental.pallas import tpu as pltpu
from jax.experimental.pallas import tpu_sc as plsc

_NUM_CORES = 2
_NUM_SUBCORES = 16
_NUM_WORKERS = _NUM_CORES * _NUM_SUBCORES
_LANES = 16
_BT = 128
_NBUF = 3


@functools.lru_cache(maxsize=None)
def _make_lookup(V, D, B0, S):
    assert B0 % (_BT * _NUM_WORKERS) == 0 or B0 == _BT * _NUM_WORKERS
    assert B0 == _BT * _NUM_WORKERS, "one output tile-column per worker"
    assert D % 8 == 0
    DT = D // 8
    scale = float(math.sqrt(D))
    per_w = _BT * S
    mesh = plsc.VectorSubcoreMesh(core_axis_name="c", subcore_axis_name="s")

    @functools.partial(
        pl.kernel,
        mesh=mesh,
        out_type=jax.ShapeDtypeStruct((S, DT, _NUM_WORKERS, 8, _BT),
                                      jnp.float32),
        scratch_types=[
            pltpu.VMEM((per_w,), jnp.int32),
            pltpu.VMEM((_NBUF, _BT), jnp.int32),
            pltpu.VMEM((_NBUF * _BT, D), jnp.float32),
            pltpu.VMEM((_NBUF * D, _BT), jnp.float32),
            pltpu.SemaphoreType.DMA,
            pltpu.SemaphoreType.DMA,
        ],
        compiler_params=pltpu.CompilerParams(
            use_tc_tiling_on_sc=False, needs_layout_passes=False
        ),
    )
    def lookup(table_hbm, idx_hbm, out_hbm, idx_v, gidx, rows, stage,
               gsem, ssem):
        wid = lax.axis_index("s") * _NUM_CORES + lax.axis_index("c")
        base = wid * per_w
        pltpu.sync_copy(idx_hbm.at[pl.ds(base, per_w)], idx_v)

        iota = lax.broadcasted_iota(jnp.int32, (_LANES,), 0)
        iota_s = iota * S
        lane_d = iota * D
        zerov = jnp.zeros((_LANES,), jnp.int32)

        def build_gidx(s, slot):
            for j in range(_BT // _LANES):
                offv = iota_s + (j * _LANES * S + s)
                vals = plsc.load_gather(idx_v, [offv])
                gidx[slot, pl.ds(j * _LANES, _LANES)] = vals

        def gather_copy(slot):
            return pltpu.make_async_copy(
                table_hbm.at[gidx.at[slot]],
                rows.at[pl.ds(slot * _BT, _BT)],
                gsem,
            )

        def scatter_copies(s, slot):
            return [
                pltpu.make_async_copy(
                    stage.at[pl.ds(slot * D + dt * 8, 8)],
                    out_hbm.at[s, dt, wid],
                    ssem,
                )
                for dt in range(DT)
            ]

        def transpose_scale(slot):
            rbase = slot * _BT * D

            def d_body(d, carry):
                for b0 in range(_BT // _LANES):
                    offv = lane_d + (rbase + b0 * _LANES * D + d)
                    v = plsc.load_gather(rows, [zerov, offv])
                    stage[slot * D + d, pl.ds(b0 * _LANES, _LANES)] = (
                        v * scale
                    )
                return carry

            lax.fori_loop(0, D, d_body, 0)

        build_gidx(0, 0)
        gather_copy(0).start()

        def body(s, carry):
            slot = lax.rem(s, _NBUF)
            nslot = lax.rem(s + 1, _NBUF)

            @pl.when(s >= 2)
            def _():
                for c in scatter_copies(s - 2, lax.rem(s - 2, _NBUF)):
                    c.wait()

            @pl.when(s + 1 <= S - 1)
            def _():
                build_gidx(s + 1, nslot)
                gather_copy(nslot).start()

            gather_copy(slot).wait()
            transpose_scale(slot)
            for c in scatter_copies(s, slot):
                c.start()
            return carry

        lax.fori_loop(0, S, body, 0)
        for c in scatter_copies(S - 2, lax.rem(S - 2, _NBUF)):
            c.wait()
        for c in scatter_copies(S - 1, lax.rem(S - 1, _NBUF)):
            c.wait()

    return lookup


def kernel(word_indices, embedding_matrix):
    B0, S = word_indices.shape
    V, D = embedding_matrix.shape
    idx = word_indices.reshape(B0 * S).astype(jnp.int32)
    lookup = _make_lookup(V, D, B0, S)
    out5 = lookup(embedding_matrix, idx)
    out = out5.transpose(2, 4, 0, 1, 3).reshape(B0, S, D)
    return out
